# baseline (speedup 1.0000x reference)
import functools

import jax
import jax.numpy as jnp
from jax import lax
from jax.experimental import pallas as pl
from jax.experimental.pallas import tpu as pltpu
from jax.experimental.pallas import tpu_sc as plsc

_NC = 2
_NS = 16
_NW = _NC * _NS

_GATHER = 128
_GPG = 2
_PAIRS = _GATHER * _GPG


def _table_body(tbl_ref, w_ref, b_ref, oa_ref, ob_ref):
    tbl = tbl_ref[...]
    n, d = tbl.shape
    rid = lax.broadcasted_iota(jnp.int32, tbl.shape, 0)
    tbl = jnp.where(rid == 0, jnp.float32(0.0), tbl)
    y = lax.dot_general(tbl, w_ref[...], (((1,), (1,)), ((), ())),
                        preferred_element_type=jnp.float32)
    t = jnp.tanh(y + b_ref[...])
    z = jnp.zeros((n, d), jnp.float32)
    oa_ref[...] = jnp.concatenate([t, z], axis=1)
    ob_ref[...] = jnp.concatenate([z, t], axis=1)


def _transform_table(emb_table, W, b):
    n, d = emb_table.shape
    return pl.pallas_call(
        _table_body,
        out_shape=[jax.ShapeDtypeStruct((n, 2 * d), jnp.float32)] * 2,
    )(emb_table, W, b.reshape(1, d))


def _gather_body(n_super, ta_hbm, tb_hbm, idx_e_hbm, idx_o_hbm, out_hbm,
                 idx_e0, idx_e1, idx_o0, idx_o1, rows_v,
                 gsem, ssem0, ssem1, isem0, isem1):
    wid = lax.axis_index("s") * _NC + lax.axis_index("c")
    pairs_w = n_super * 2 * _PAIRS
    base = wid * pairs_w
    ssems = (ssem0, ssem1)
    isems = (isem0, isem1)
    idx_es = (idx_e0, idx_e1)
    idx_os = (idx_o0, idx_o1)

    def idx_load(g, s):
        off = base + g * _PAIRS
        pltpu.async_copy(idx_e_hbm.at[pl.ds(off, _PAIRS)], idx_es[s], isems[s])
        pltpu.async_copy(idx_o_hbm.at[pl.ds(off, _PAIRS)], idx_os[s], isems[s])

    for s in range(2):
        idx_load(s, s)

    def super_body(t, carry):
        for s in range(2):
            g = 2 * t + s
            off = base + g * _PAIRS

            @pl.when(t > 0)
            def _wait_store(s=s):
                pltpu.make_async_copy(
                    rows_v.at[s], out_hbm.at[pl.ds(0, _PAIRS)],
                    ssems[s]).wait()

            for r in (idx_es[s], idx_os[s]):
                pltpu.make_async_copy(
                    idx_e_hbm.at[pl.ds(0, _PAIRS)], r, isems[s]).wait()

            cps = [
                pltpu.async_copy(
                    ta_hbm.at[idx_es[s].at[pl.ds(j * _GATHER, _GATHER)]],
                    rows_v.at[s, pl.ds(j * _GATHER, _GATHER), :],
                    gsem)
                for j in range(_GPG)
            ]
            for cp in cps:
                cp.wait()
            cps = [
                pltpu.async_copy(
                    tb_hbm.at[idx_os[s].at[pl.ds(j * _GATHER, _GATHER)]],
                    rows_v.at[s, pl.ds(j * _GATHER, _GATHER), :],
                    gsem, add=True)
                for j in range(_GPG)
            ]
            for cp in cps:
                cp.wait()

            @pl.when(g + 2 < 2 * n_super)
            def _prefetch(g=g, s=s):
                idx_load(g + 2, s)

            pltpu.async_copy(
                rows_v.at[s], out_hbm.at[pl.ds(off, _PAIRS)], ssems[s])
        return carry

    lax.fori_loop(0, n_super, super_body, 0)
    for s in range(2):
        pltpu.make_async_copy(
            rows_v.at[s], out_hbm.at[pl.ds(0, _PAIRS)], ssems[s]).wait()


def _gather_rows(ta, tb, idx_e, idx_o):
    half = idx_e.shape[0]
    d2 = ta.shape[1]
    assert half % (_NW * 2 * _PAIRS) == 0
    n_super = half // (_NW * 2 * _PAIRS)

    mesh = plsc.VectorSubcoreMesh(core_axis_name="c", subcore_axis_name="s")
    kern = pl.kernel(
        functools.partial(_gather_body, n_super),
        mesh=mesh,
        out_type=jax.ShapeDtypeStruct((half, d2), jnp.float32),
        scratch_types=[
            pltpu.VMEM((_PAIRS,), jnp.int32),
            pltpu.VMEM((_PAIRS,), jnp.int32),
            pltpu.VMEM((_PAIRS,), jnp.int32),
            pltpu.VMEM((_PAIRS,), jnp.int32),
            pltpu.VMEM((2, _PAIRS, d2), jnp.float32),
            pltpu.SemaphoreType.DMA,
            pltpu.SemaphoreType.DMA,
            pltpu.SemaphoreType.DMA,
            pltpu.SemaphoreType.DMA,
            pltpu.SemaphoreType.DMA,
        ],
    )
    return kern(ta, tb, idx_e, idx_o)


def kernel(sentiment, emb_table, W, b):
    batch, hist = sentiment.shape
    d = emb_table.shape[1]
    ta, tb = _transform_table(emb_table, W, b)
    idx2 = sentiment.reshape(-1, 2)
    out2 = _gather_rows(ta, tb, idx2[:, 0], idx2[:, 1])
    return out2.reshape(batch, hist, d)


# speedup vs baseline: 1.1644x; 1.1644x over previous
import functools

import jax
import jax.numpy as jnp
from jax import lax
from jax.experimental import pallas as pl
from jax.experimental.pallas import tpu as pltpu
from jax.experimental.pallas import tpu_sc as plsc

_NC = 2
_NS = 16
_NW = _NC * _NS

_GATHER = 128
_GPG = 2
_PAIRS = _GATHER * _GPG


def _table_body(tbl_ref, w_ref, b_ref, oa_ref, ob_ref):
    tbl = tbl_ref[...]
    n, d = tbl.shape
    rid = lax.broadcasted_iota(jnp.int32, tbl.shape, 0)
    tbl = jnp.where(rid == 0, jnp.float32(0.0), tbl)
    y = lax.dot_general(tbl, w_ref[...], (((1,), (1,)), ((), ())),
                        preferred_element_type=jnp.float32)
    t = jnp.tanh(y + b_ref[...])
    z = jnp.zeros((n, d), jnp.float32)
    oa_ref[...] = jnp.concatenate([t, z], axis=1)
    ob_ref[...] = jnp.concatenate([z, t], axis=1)


def _transform_table(emb_table, W, b):
    n, d = emb_table.shape
    return pl.pallas_call(
        _table_body,
        out_shape=[jax.ShapeDtypeStruct((n, 2 * d), jnp.float32)] * 2,
    )(emb_table, W, b.reshape(1, d))


def _gather_body(n_super, ta_hbm, tb_hbm, idx_e_hbm, idx_o_hbm, out_hbm,
                 idx_e0, idx_e1, idx_o0, idx_o1, rows_v, ta_s, tb_s,
                 gsem, ssem0, ssem1, isem0, isem1):
    wid = lax.axis_index("s") * _NC + lax.axis_index("c")
    pairs_w = n_super * 2 * _PAIRS
    base = wid * pairs_w
    ssems = (ssem0, ssem1)
    isems = (isem0, isem1)
    idx_es = (idx_e0, idx_e1)
    idx_os = (idx_o0, idx_o1)

    def idx_load(g, s):
        off = base + g * _PAIRS
        pltpu.async_copy(idx_e_hbm.at[pl.ds(off, _PAIRS)], idx_es[s], isems[s])
        pltpu.async_copy(idx_o_hbm.at[pl.ds(off, _PAIRS)], idx_os[s], isems[s])

    for s in range(2):
        idx_load(s, s)

    @pl.when(lax.axis_index("s") == 0)
    def _stage():
        pltpu.sync_copy(ta_hbm, ta_s)
        pltpu.sync_copy(tb_hbm, tb_s)

    plsc.subcore_barrier()

    def super_body(t, carry):
        for s in range(2):
            g = 2 * t + s
            off = base + g * _PAIRS

            @pl.when(t > 0)
            def _wait_store(s=s):
                pltpu.make_async_copy(
                    rows_v.at[s], out_hbm.at[pl.ds(0, _PAIRS)],
                    ssems[s]).wait()

            for r in (idx_es[s], idx_os[s]):
                pltpu.make_async_copy(
                    idx_e_hbm.at[pl.ds(0, _PAIRS)], r, isems[s]).wait()

            cps = [
                pltpu.async_copy(
                    ta_s.at[idx_es[s].at[pl.ds(j * _GATHER, _GATHER)]],
                    rows_v.at[s, pl.ds(j * _GATHER, _GATHER), :],
                    gsem)
                for j in range(_GPG)
            ]
            for cp in cps:
                cp.wait()
            cps = [
                pltpu.async_copy(
                    tb_s.at[idx_os[s].at[pl.ds(j * _GATHER, _GATHER)]],
                    rows_v.at[s, pl.ds(j * _GATHER, _GATHER), :],
                    gsem, add=True)
                for j in range(_GPG)
            ]
            for cp in cps:
                cp.wait()

            @pl.when(g + 2 < 2 * n_super)
            def _prefetch(g=g, s=s):
                idx_load(g + 2, s)

            pltpu.async_copy(
                rows_v.at[s], out_hbm.at[pl.ds(off, _PAIRS)], ssems[s])
        return carry

    lax.fori_loop(0, n_super, super_body, 0)
    for s in range(2):
        pltpu.make_async_copy(
            rows_v.at[s], out_hbm.at[pl.ds(0, _PAIRS)], ssems[s]).wait()


def _gather_rows(ta, tb, idx_e, idx_o):
    half = idx_e.shape[0]
    d2 = ta.shape[1]
    assert half % (_NW * 2 * _PAIRS) == 0
    n_super = half // (_NW * 2 * _PAIRS)

    mesh = plsc.VectorSubcoreMesh(core_axis_name="c", subcore_axis_name="s")
    kern = pl.kernel(
        functools.partial(_gather_body, n_super),
        mesh=mesh,
        out_type=jax.ShapeDtypeStruct((half, d2), jnp.float32),
        scratch_types=[
            pltpu.VMEM((_PAIRS,), jnp.int32),
            pltpu.VMEM((_PAIRS,), jnp.int32),
            pltpu.VMEM((_PAIRS,), jnp.int32),
            pltpu.VMEM((_PAIRS,), jnp.int32),
            pltpu.VMEM((2, _PAIRS, d2), jnp.float32),
            pltpu.VMEM_SHARED((ta.shape[0], d2), jnp.float32),
            pltpu.VMEM_SHARED((ta.shape[0], d2), jnp.float32),
            pltpu.SemaphoreType.DMA,
            pltpu.SemaphoreType.DMA,
            pltpu.SemaphoreType.DMA,
            pltpu.SemaphoreType.DMA,
            pltpu.SemaphoreType.DMA,
        ],
    )
    return kern(ta, tb, idx_e, idx_o)


def kernel(sentiment, emb_table, W, b):
    batch, hist = sentiment.shape
    d = emb_table.shape[1]
    ta, tb = _transform_table(emb_table, W, b)
    idx2 = sentiment.reshape(-1, 2)
    out2 = _gather_rows(ta, tb, idx2[:, 0], idx2[:, 1])
    return out2.reshape(batch, hist, d)


# speedup vs baseline: 1.7621x; 1.5133x over previous
import functools

import jax
import jax.numpy as jnp
from jax import lax
from jax.experimental import pallas as pl
from jax.experimental.pallas import tpu as pltpu
from jax.experimental.pallas import tpu_sc as plsc

_NC = 2
_NS = 16
_NW = _NC * _NS

_HIST = 200
_PB = 112
_GPB = 2
_IDXG = _GPB * _HIST
_ROWS = _GPB * _PB
_SLAB = _IDXG + 32
_SPLITS = ((0, 128), (128, 96))


def _table_body(tbl_ref, w_ref, b_ref, oa_ref, ob_ref):
    tbl = tbl_ref[...]
    n, d = tbl.shape
    rid = lax.broadcasted_iota(jnp.int32, tbl.shape, 0)
    tbl = jnp.where(rid == 0, jnp.float32(0.0), tbl)
    y = lax.dot_general(tbl, w_ref[...], (((1,), (1,)), ((), ())),
                        preferred_element_type=jnp.float32)
    t = jnp.tanh(y + b_ref[...])
    z = jnp.zeros((n, d), jnp.float32)
    oa_ref[...] = jnp.concatenate([t, z], axis=1)
    ob_ref[...] = jnp.concatenate([z, t], axis=1)


def _transform_table(emb_table, W, b):
    n, d = emb_table.shape
    return pl.pallas_call(
        _table_body,
        out_shape=[jax.ShapeDtypeStruct((n, 2 * d), jnp.float32)] * 2,
    )(emb_table, W, b.reshape(1, d))


def _gather_body(n_super, ta_hbm, tb_hbm, idx_hbm, out_hbm,
                 slab0, slab1, ie0, ie1, io0, io1, rows_v, ta_s, tb_s,
                 gsem, ssem0, ssem1, isem0, isem1):
    wid = lax.axis_index("s") * _NC + lax.axis_index("c")
    groups_w = n_super * 2
    base_g = wid * groups_w
    ssems = (ssem0, ssem1)
    isems = (isem0, isem1)
    slabs = (slab0, slab1)
    ies = (ie0, ie1)
    ios = (io0, io1)

    iota = lax.iota(jnp.int32, 16)
    iota2 = iota * 2
    zeros16 = jnp.zeros((16,), jnp.int32)

    for slab in slabs:
        for k in range(_SLAB - _IDXG, 0, -16):
            slab[pl.ds(_SLAB - k, 16)] = zeros16

    def idx_load(g, s):
        pltpu.async_copy(idx_hbm.at[pl.ds((base_g + g) * _IDXG, _IDXG)],
                         slabs[s].at[pl.ds(0, _IDXG)], isems[s])

    for s in range(2):
        idx_load(s, s)

    @pl.when(lax.axis_index("s") == 0)
    def _stage():
        pltpu.sync_copy(ta_hbm, ta_s)
        pltpu.sync_copy(tb_hbm, tb_s)

    plsc.subcore_barrier()

    def super_body(t, carry):
        for s in range(2):
            g = 2 * t + s

            @pl.when(t > 0)
            def _wait_store(s=s):
                pltpu.make_async_copy(
                    rows_v.at[s], out_hbm.at[pl.ds(0, _ROWS)],
                    ssems[s]).wait()

            pltpu.make_async_copy(
                idx_hbm.at[pl.ds(0, _IDXG)],
                slabs[s].at[pl.ds(0, _IDXG)], isems[s]).wait()

            for b in range(_GPB):
                for k in range(_PB // 16):
                    m0 = 16 * k
                    src = b * _HIST + 2 * m0
                    ev = plsc.load_gather(slabs[s], [iota2 + src])
                    od = plsc.load_gather(slabs[s], [iota2 + (src + 1)])
                    ies[s][pl.ds(b * _PB + m0, 16)] = ev
                    ios[s][pl.ds(b * _PB + m0, 16)] = od

            cps = [
                pltpu.async_copy(
                    ta_s.at[ies[s].at[pl.ds(off, n)]],
                    rows_v.at[s, pl.ds(off, n), :], gsem)
                for off, n in _SPLITS
            ]
            for cp in cps:
                cp.wait()
            cps = [
                pltpu.async_copy(
                    tb_s.at[ios[s].at[pl.ds(off, n)]],
                    rows_v.at[s, pl.ds(off, n), :], gsem, add=True)
                for off, n in _SPLITS
            ]
            for cp in cps:
                cp.wait()

            @pl.when(g + 2 < groups_w)
            def _prefetch(g=g, s=s):
                idx_load(g + 2, s)

            pltpu.async_copy(
                rows_v.at[s],
                out_hbm.at[pl.ds((base_g + g) * _ROWS, _ROWS)], ssems[s])
        return carry

    lax.fori_loop(0, n_super, super_body, 0)
    for s in range(2):
        pltpu.make_async_copy(
            rows_v.at[s], out_hbm.at[pl.ds(0, _ROWS)], ssems[s]).wait()


def _gather_rows(ta, tb, idx, batch):
    d2 = ta.shape[1]
    assert batch % (_NW * 2 * _GPB) == 0
    n_super = batch // (_NW * 2 * _GPB)

    mesh = plsc.VectorSubcoreMesh(core_axis_name="c", subcore_axis_name="s")
    kern = pl.kernel(
        functools.partial(_gather_body, n_super),
        mesh=mesh,
        compiler_params=pltpu.CompilerParams(needs_layout_passes=False),
        out_type=jax.ShapeDtypeStruct((batch * _PB, d2), jnp.float32),
        scratch_types=[
            pltpu.VMEM((_SLAB,), jnp.int32),
            pltpu.VMEM((_SLAB,), jnp.int32),
            pltpu.VMEM((_ROWS,), jnp.int32),
            pltpu.VMEM((_ROWS,), jnp.int32),
            pltpu.VMEM((_ROWS,), jnp.int32),
            pltpu.VMEM((_ROWS,), jnp.int32),
            pltpu.VMEM((2, _ROWS, d2), jnp.float32),
            pltpu.VMEM_SHARED((ta.shape[0], d2), jnp.float32),
            pltpu.VMEM_SHARED((ta.shape[0], d2), jnp.float32),
            pltpu.SemaphoreType.DMA,
            pltpu.SemaphoreType.DMA,
            pltpu.SemaphoreType.DMA,
            pltpu.SemaphoreType.DMA,
            pltpu.SemaphoreType.DMA,
        ],
    )
    return kern(ta, tb, idx)


def _transpose_body(in_ref, out_ref):
    for j in range(8):
        for h in range(2):
            plane = in_ref[:, j, h * 64:(h + 1) * 64]
            out_ref[2 * j + h] = plane.T


def _transpose(l3, batch, hist, d):
    grid = (pl.cdiv(hist, 16), batch // 128)
    return pl.pallas_call(
        _transpose_body,
        grid=grid,
        in_specs=[pl.BlockSpec((128, 8, 2 * d),
                               lambda q, bc: (bc, q, 0))],
        out_specs=pl.BlockSpec((16, d, 128), lambda q, bc: (q, 0, bc)),
        out_shape=jax.ShapeDtypeStruct((hist, d, batch), jnp.float32),
    )(l3)


def kernel(sentiment, emb_table, W, b):
    batch, hist = sentiment.shape
    d = emb_table.shape[1]
    ta, tb = _transform_table(emb_table, W, b)
    l2 = _gather_rows(ta, tb, sentiment.reshape(-1), batch)
    l3 = l2.reshape(batch, _PB, 2 * d)
    out_k = _transpose(l3, batch, hist, d)
    return out_k.transpose(2, 0, 1)


# speedup vs baseline: 2.1089x; 1.1968x over previous
import functools

import jax
import jax.numpy as jnp
from jax import lax
from jax.experimental import pallas as pl
from jax.experimental.pallas import tpu as pltpu
from jax.experimental.pallas import tpu_sc as plsc

_NC = 2
_NS = 16
_NW = _NC * _NS

_HIST = 200
_PB = 112
_GPB = 2
_IDXG = _GPB * _HIST
_ROWS = _GPB * _PB
_SLAB = _IDXG + 32
_SPLITS = ((0, 128), (128, 96))


def _table_body(tbl_ref, w_ref, b_ref, oa_ref, ob_ref):
    tbl = tbl_ref[...]
    n, d = tbl.shape
    rid = lax.broadcasted_iota(jnp.int32, tbl.shape, 0)
    tbl = jnp.where(rid == 0, jnp.float32(0.0), tbl)
    y = lax.dot_general(tbl, w_ref[...], (((1,), (1,)), ((), ())),
                        preferred_element_type=jnp.float32)
    t = jnp.tanh(y + b_ref[...])
    z = jnp.zeros((n, d), jnp.float32)
    oa_ref[...] = jnp.concatenate([t, z], axis=1)
    ob_ref[...] = jnp.concatenate([z, t], axis=1)


def _transform_table(emb_table, W, b):
    n, d = emb_table.shape
    return pl.pallas_call(
        _table_body,
        out_shape=[jax.ShapeDtypeStruct((n, 2 * d), jnp.float32)] * 2,
    )(emb_table, W, b.reshape(1, d))


def _gather_body(n_super, chunk_g0, ta_hbm, tb_hbm, idx_hbm, out_hbm,
                 slab0, slab1, ie0, ie1, io0, io1, rows_v, ta_s, tb_s,
                 gsem, ssem0, ssem1, isem0, isem1):
    wid = lax.axis_index("s") * _NC + lax.axis_index("c")
    groups_w = n_super * 2
    base_g = wid * groups_w
    ssems = (ssem0, ssem1)
    isems = (isem0, isem1)
    slabs = (slab0, slab1)
    ies = (ie0, ie1)
    ios = (io0, io1)

    iota = lax.iota(jnp.int32, 16)
    iota2 = iota * 2
    zeros16 = jnp.zeros((16,), jnp.int32)

    for slab in slabs:
        for k in range(_SLAB - _IDXG, 0, -16):
            slab[pl.ds(_SLAB - k, 16)] = zeros16

    def idx_load(g, s):
        pltpu.async_copy(
            idx_hbm.at[pl.ds((chunk_g0 + base_g + g) * _IDXG, _IDXG)],
            slabs[s].at[pl.ds(0, _IDXG)], isems[s])

    for s in range(2):
        idx_load(s, s)

    @pl.when(lax.axis_index("s") == 0)
    def _stage():
        pltpu.sync_copy(ta_hbm, ta_s)
        pltpu.sync_copy(tb_hbm, tb_s)

    plsc.subcore_barrier()

    def super_body(t, carry):
        for s in range(2):
            g = 2 * t + s

            @pl.when(t > 0)
            def _wait_store(s=s):
                pltpu.make_async_copy(
                    rows_v.at[s], out_hbm.at[pl.ds(0, _ROWS)],
                    ssems[s]).wait()

            pltpu.make_async_copy(
                idx_hbm.at[pl.ds(0, _IDXG)],
                slabs[s].at[pl.ds(0, _IDXG)], isems[s]).wait()

            for b in range(_GPB):
                for k in range(_PB // 16):
                    m0 = 16 * k
                    src = b * _HIST + 2 * m0
                    ev = plsc.load_gather(slabs[s], [iota2 + src])
                    od = plsc.load_gather(slabs[s], [iota2 + (src + 1)])
                    ies[s][pl.ds(b * _PB + m0, 16)] = ev
                    ios[s][pl.ds(b * _PB + m0, 16)] = od

            cps = [
                pltpu.async_copy(
                    ta_s.at[ies[s].at[pl.ds(off, n)]],
                    rows_v.at[s, pl.ds(off, n), :], gsem)
                for off, n in _SPLITS
            ]
            for cp in cps:
                cp.wait()
            cps = [
                pltpu.async_copy(
                    tb_s.at[ios[s].at[pl.ds(off, n)]],
                    rows_v.at[s, pl.ds(off, n), :], gsem, add=True)
                for off, n in _SPLITS
            ]
            for cp in cps:
                cp.wait()

            @pl.when(g + 2 < groups_w)
            def _prefetch(g=g, s=s):
                idx_load(g + 2, s)

            pltpu.async_copy(
                rows_v.at[s],
                out_hbm.at[pl.ds((base_g + g) * _ROWS, _ROWS)], ssems[s])
        return carry

    lax.fori_loop(0, n_super, super_body, 0)
    for s in range(2):
        pltpu.make_async_copy(
            rows_v.at[s], out_hbm.at[pl.ds(0, _ROWS)], ssems[s]).wait()


def _gather_rows(ta, tb, idx, batch_c, chunk_g0):
    d2 = ta.shape[1]
    assert batch_c % (_NW * 2 * _GPB) == 0
    n_super = batch_c // (_NW * 2 * _GPB)

    mesh = plsc.VectorSubcoreMesh(core_axis_name="c", subcore_axis_name="s")
    kern = pl.kernel(
        functools.partial(_gather_body, n_super, chunk_g0),
        mesh=mesh,
        compiler_params=pltpu.CompilerParams(needs_layout_passes=False),
        out_type=jax.ShapeDtypeStruct((batch_c * _PB, d2), jnp.float32),
        scratch_types=[
            pltpu.VMEM((_SLAB,), jnp.int32),
            pltpu.VMEM((_SLAB,), jnp.int32),
            pltpu.VMEM((_ROWS,), jnp.int32),
            pltpu.VMEM((_ROWS,), jnp.int32),
            pltpu.VMEM((_ROWS,), jnp.int32),
            pltpu.VMEM((_ROWS,), jnp.int32),
            pltpu.VMEM((2, _ROWS, d2), jnp.float32),
            pltpu.VMEM_SHARED((ta.shape[0], d2), jnp.float32),
            pltpu.VMEM_SHARED((ta.shape[0], d2), jnp.float32),
            pltpu.SemaphoreType.DMA,
            pltpu.SemaphoreType.DMA,
            pltpu.SemaphoreType.DMA,
            pltpu.SemaphoreType.DMA,
            pltpu.SemaphoreType.DMA,
        ],
    )
    return kern(ta, tb, idx)


def _transpose_body(in_ref, out_ref):
    for j in range(8):
        for h in range(2):
            plane = in_ref[:, j, h * 64:(h + 1) * 64]
            out_ref[2 * j + h] = plane.T


def _transpose_acc_body(in_ref, prev_ref, out_ref):
    del prev_ref
    _transpose_body(in_ref, out_ref)


def _transpose_chunk(l3, prev, batch, hist, d, c, nbc):
    grid = (pl.cdiv(hist, 16), nbc)
    in_spec = pl.BlockSpec((128, 8, 2 * d), lambda q, bc: (bc, q, 0))
    out_spec = pl.BlockSpec((16, d, 128), lambda q, bc: (q, 0, c * nbc + bc))
    out_shape = jax.ShapeDtypeStruct((hist, d, batch), jnp.float32)
    if prev is None:
        return pl.pallas_call(
            _transpose_body,
            grid=grid,
            in_specs=[in_spec],
            out_specs=out_spec,
            out_shape=out_shape,
        )(l3)
    return pl.pallas_call(
        _transpose_acc_body,
        grid=grid,
        in_specs=[in_spec, pl.BlockSpec(memory_space=pl.ANY)],
        out_specs=out_spec,
        out_shape=out_shape,
        input_output_aliases={1: 0},
    )(l3, prev)


_CHUNKS = 4


def kernel(sentiment, emb_table, W, b):
    batch, hist = sentiment.shape
    d = emb_table.shape[1]
    ta, tb = _transform_table(emb_table, W, b)
    idx = sentiment.reshape(-1)
    batch_c = batch // _CHUNKS
    nbc = batch_c // 128
    out_k = None
    for c in range(_CHUNKS):
        l2 = _gather_rows(ta, tb, idx, batch_c,
                          c * (batch_c // _GPB))
        l3 = l2.reshape(batch_c, _PB, 2 * d)
        out_k = _transpose_chunk(l3, out_k, batch, hist, d, c, nbc)
    return out_k.transpose(2, 0, 1)
